# Initial kernel scaffold; baseline (speedup 1.0000x reference)
import jax
import jax.numpy as jnp
from jax.experimental import pallas as pl


def kernel(coords, atom_types, radii, query_points, atom_embed):
    raise NotImplementedError("write your pallas kernel here")


# speedup vs baseline: 4.1502x; 4.1502x over previous
import jax
import jax.numpy as jnp
from jax import lax
from jax.experimental import pallas as pl

_CUTOFF = 6.0
_K = 32
_RBF = 32
_EDIM = 64
_GAMMA = 1.0 / max(_CUTOFF / max(_RBF, 1), 1e-06) ** 2
_GL = 32
_NG = 128
_FDIM = 3 + 1 + _EDIM + _RBF + 1


def _fb_kernel(q_ref, c_ref, a_ref, e_ref, cen_ref,
               feat_ref, mask_ref, idx_ref, dist_ref):
  qt = q_ref.shape[1]
  n = c_ref.shape[2]
  q = q_ref[0]
  qx, qy, qz = q[:, 0:1], q[:, 1:2], q[:, 2:3]
  cx = c_ref[0, 0:1, :]
  cy = c_ref[0, 1:2, :]
  cz = c_ref[0, 2:3, :]
  dx = qx - cx
  dy = qy - cy
  dz = qz - cz
  d = jnp.sqrt(dx * dx + dy * dy + dz * dz + 1e-12)

  lane = lax.broadcasted_iota(jnp.int32, (qt, n), 1)
  col = lax.broadcasted_iota(jnp.int32, (qt, _K), 1)
  ng = a_ref.shape[1]
  gi = lax.broadcasted_iota(jnp.int32, (qt, ng), 1)
  li = lax.broadcasted_iota(jnp.int32, (qt, _GL), 1)
  ei = lax.broadcasted_iota(jnp.int32, (qt, 128), 1)
  attr = a_ref[0]
  etab = e_ref[...]
  cen = cen_ref[0:1, :]

  work = d
  top_d = jnp.zeros((qt, _K), jnp.float32)
  top_i = jnp.zeros((qt, _K), jnp.int32)
  inf = jnp.float32(jnp.inf)
  for j in range(_K):
    m = jnp.min(work, axis=1, keepdims=True)
    ij = jnp.min(jnp.where(work == m, lane, n), axis=1, keepdims=True)
    work = jnp.where(lane == ij, inf, work)
    g = ij // _GL
    l = ij - g * _GL
    oh_g = (g == gi).astype(jnp.float32)
    st1 = jnp.dot(oh_g, attr,
                  preferred_element_type=jnp.float32)
    oh_l = l == li

    def sel(s):
      seg = st1[:, s * _GL:(s + 1) * _GL]
      return jnp.sum(jnp.where(oh_l, seg, 0.0), axis=1, keepdims=True)

    cxj, cyj, czj, radj, typj = sel(0), sel(1), sel(2), sel(3), sel(4)
    oh_e = (typj.astype(jnp.int32) == ei).astype(jnp.float32)
    embj = jnp.dot(oh_e, etab,
                   preferred_element_type=jnp.float32)
    rbfj = jnp.exp(-_GAMMA * (m - cen) ** 2)
    featj = jnp.concatenate(
        [qx - cxj, qy - cyj, qz - czj, radj, embj, rbfj, m], axis=1)
    feat_ref[0, j] = jnp.where(m <= _CUTOFF, featj, 0.0)
    cj = col == j
    top_d = jnp.where(cj, m, top_d)
    top_i = jnp.where(cj, ij, top_i)

  maskqk = top_d <= _CUTOFF
  mask_ref[0] = maskqk.astype(jnp.int32)
  idx_ref[0] = top_i
  dist_ref[0] = jnp.where(maskqk, top_d, 0.0)


def kernel(coords, atom_types, radii, query_points, atom_embed):
  b, n, _ = coords.shape
  _, q, _ = query_points.shape
  qt = min(128, q)
  grid = (b, q // qt)
  ng = n // _GL
  coords_t = coords.transpose(0, 2, 1)
  cxg = coords[..., 0].reshape(b, ng, _GL)
  cyg = coords[..., 1].reshape(b, ng, _GL)
  czg = coords[..., 2].reshape(b, ng, _GL)
  rg = radii.reshape(b, ng, _GL)
  tg = atom_types.astype(jnp.float32).reshape(b, ng, _GL)
  attr = jnp.concatenate([cxg, cyg, czg, rg, tg], axis=-1)
  epad = jnp.zeros((128, _EDIM), jnp.float32)
  epad = epad.at[: atom_embed.shape[0]].set(atom_embed.astype(jnp.float32))
  centers = jnp.broadcast_to(
      jnp.linspace(0.0, _CUTOFF, _RBF).astype(jnp.float32)[None, :],
      (8, _RBF))
  feat_t, mask_i, idx, dist = pl.pallas_call(
      _fb_kernel,
      grid=grid,
      in_specs=[
          pl.BlockSpec((1, qt, 3), lambda bb, ii: (bb, ii, 0)),
          pl.BlockSpec((1, 3, n), lambda bb, ii: (bb, 0, 0)),
          pl.BlockSpec((1, ng, 5 * _GL), lambda bb, ii: (bb, 0, 0)),
          pl.BlockSpec((128, _EDIM), lambda bb, ii: (0, 0)),
          pl.BlockSpec((8, _RBF), lambda bb, ii: (0, 0)),
      ],
      out_specs=[
          pl.BlockSpec((1, _K, qt, _FDIM), lambda bb, ii: (bb, 0, ii, 0)),
          pl.BlockSpec((1, qt, _K), lambda bb, ii: (bb, ii, 0)),
          pl.BlockSpec((1, qt, _K), lambda bb, ii: (bb, ii, 0)),
          pl.BlockSpec((1, qt, _K), lambda bb, ii: (bb, ii, 0)),
      ],
      out_shape=[
          jax.ShapeDtypeStruct((b, _K, q, _FDIM), jnp.float32),
          jax.ShapeDtypeStruct((b, q, _K), jnp.int32),
          jax.ShapeDtypeStruct((b, q, _K), jnp.int32),
          jax.ShapeDtypeStruct((b, q, _K), jnp.float32),
      ],
  )(query_points, coords_t, attr, epad, centers)
  feat = feat_t.transpose(0, 2, 1, 3)
  return feat, mask_i.astype(bool), idx, dist
